# Initial kernel scaffold; baseline (speedup 1.0000x reference)
import jax
import jax.numpy as jnp
from jax.experimental import pallas as pl


def kernel(x, edge_index, W, b):
    raise NotImplementedError("write your pallas kernel here")


# speedup vs baseline: 8.0119x; 8.0119x over previous
import functools

import jax
import jax.numpy as jnp
from jax import lax
from jax.experimental import pallas as pl
from jax.experimental.pallas import tpu as pltpu
from jax.experimental.pallas import tpu_sc as plsc

N_NODES = 10000
D = 128
EPS = 1e-07

NS = 16
CHUNK = 128
GROUP = 32
TILE_ROWS = 640
ACC_ROWS = NS * TILE_ROWS

ROW_BLK = 1000


def _table_body(x_ref, u_ref, t_ref):
    m = jnp.maximum(x_ref[...], 0.0) + EPS
    t = jnp.exp(m)
    u_ref[...] = m * t
    t_ref[...] = t


def _final_body(num_ref, den_ref, w_ref, b_ref, o_ref):
    den = den_ref[0]
    a = jnp.where(den > 0.0, num_ref[0] / den, 0.0)
    o_ref[...] = (
        lax.dot_general(a, w_ref[...], (((1,), (1,)), ((), ())),
                        preferred_element_type=jnp.float32)
        + b_ref[...]
    )


def _sc_edge_body(n_chunks, u_hbm, t_hbm, src_hbm, dst_hbm, out_hbm, acc):
    pl.run_scoped(
        functools.partial(_sc_edge_inner, n_chunks, u_hbm, t_hbm, src_hbm,
                          dst_hbm, out_hbm, acc),
        pltpu.VMEM((GROUP, CHUNK), jnp.int32),
        pltpu.VMEM((GROUP, CHUNK), jnp.int32),
        pltpu.VMEM((CHUNK, D), jnp.float32),
        pltpu.VMEM((CHUNK, D), jnp.float32),
        pltpu.SemaphoreType.DMA,
        pltpu.SemaphoreType.DMA,
    )


def _sc_edge_inner(n_chunks, u_hbm, t_hbm, src_hbm, dst_hbm, out_hbm, acc,
                   src_v, dst_v, rows_a, rows_b, sem_a, sem_b):
    c = lax.axis_index("c")
    s = lax.axis_index("s")
    base = s * TILE_ROWS

    def _zero_row(r, carry):
        for k in range(8):
            rows_a[r, pl.ds(16 * k, 16)] = jnp.zeros((16,), jnp.float32)
        return carry

    lax.fori_loop(0, CHUNK, _zero_row, 0)
    for k in range(TILE_ROWS // CHUNK):
        pltpu.sync_copy(rows_a, acc.at[pl.ds(base + k * CHUNK, CHUNK)])

    plsc.subcore_barrier()

    def _run(tab):
        def _start(j, rows, sem):
            pltpu.make_async_copy(tab.at[src_v.at[j]], rows, sem).start()

        def _finish(j, rows, sem):
            pltpu.make_async_copy(tab.at[src_v.at[j]], rows, sem).wait()
            pltpu.sync_copy(rows, acc.at[dst_v.at[j]], add=True)

        def _group(g, carry):
            pltpu.sync_copy(src_hbm.at[s, pl.ds(g * GROUP, GROUP)], src_v)
            pltpu.sync_copy(dst_hbm.at[s, pl.ds(g * GROUP, GROUP)], dst_v)
            _start(0, rows_a, sem_a)

            def _pair(i, cc):
                j0 = 2 * i
                j1 = j0 + 1
                _start(j1, rows_b, sem_b)
                _finish(j0, rows_a, sem_a)

                @pl.when(j1 + 1 < GROUP)
                def _():
                    _start(j1 + 1, rows_a, sem_a)

                _finish(j1, rows_b, sem_b)
                return cc

            lax.fori_loop(0, GROUP // 2, _pair, 0)
            return carry

        lax.fori_loop(0, n_chunks // GROUP, _group, 0)

    @pl.when(c == 0)
    def _():
        _run(u_hbm)

    @pl.when(c == 1)
    def _():
        _run(t_hbm)

    plsc.subcore_barrier()

    for k in range(TILE_ROWS // CHUNK):
        off = base + k * CHUNK
        pltpu.sync_copy(acc.at[pl.ds(off, CHUNK)], rows_a)
        pltpu.sync_copy(rows_a, out_hbm.at[c, pl.ds(off, CHUNK)])


def kernel(x, edge_index, W, b):
    n = x.shape[0]
    e = edge_index.shape[1]

    grid = n // ROW_BLK
    u, t = pl.pallas_call(
        _table_body,
        grid=(grid,),
        in_specs=[pl.BlockSpec((ROW_BLK, D), lambda i: (i, 0))],
        out_specs=[pl.BlockSpec((ROW_BLK, D), lambda i: (i, 0)),
                   pl.BlockSpec((ROW_BLK, D), lambda i: (i, 0))],
        out_shape=[jax.ShapeDtypeStruct((n, D), jnp.float32),
                   jax.ShapeDtypeStruct((n, D), jnp.float32)],
    )(x)

    n_groups = -(-e // (NS * CHUNK * GROUP))
    n_chunks = n_groups * GROUP
    e_pad = NS * n_chunks * CHUNK
    dst = edge_index[0]
    src = edge_index[1]
    if e_pad > e:
        pad = e_pad - e
        src = jnp.concatenate([src, jnp.zeros((pad,), jnp.int32)])
        dst = jnp.concatenate([dst, jnp.full((pad,), N_NODES, jnp.int32)])
    src_r = src.reshape(NS, n_chunks, CHUNK)
    dst_r = dst.reshape(NS, n_chunks, CHUNK)

    mesh = plsc.VectorSubcoreMesh(core_axis_name="c", subcore_axis_name="s")
    nd = pl.kernel(
        functools.partial(_sc_edge_body, n_chunks),
        mesh=mesh,
        out_type=jax.ShapeDtypeStruct((2, ACC_ROWS, D), jnp.float32),
        scratch_types=[
            pltpu.VMEM_SHARED((ACC_ROWS, D), jnp.float32),
        ],
    )(u, t, src_r, dst_r)

    out = pl.pallas_call(
        _final_body,
        grid=(grid,),
        in_specs=[pl.BlockSpec((1, ROW_BLK, D), lambda i: (0, i, 0)),
                  pl.BlockSpec((1, ROW_BLK, D), lambda i: (1, i, 0)),
                  pl.BlockSpec((D, D), lambda i: (0, 0)),
                  pl.BlockSpec((1, D), lambda i: (0, 0))],
        out_specs=pl.BlockSpec((ROW_BLK, D), lambda i: (i, 0)),
        out_shape=jax.ShapeDtypeStruct((n, D), jnp.float32),
    )(nd, nd, W, b.reshape(1, D))
    return out
